# Initial kernel scaffold; baseline (speedup 1.0000x reference)
import jax
import jax.numpy as jnp
from jax.experimental import pallas as pl


def kernel(x, edge_index, train_edge_id, W1, b1, W2, b2, W3, b3, W4, b4):
    raise NotImplementedError("write your pallas kernel here")


# speedup vs baseline: 2.5446x; 2.5446x over previous
import functools

import jax
import jax.numpy as jnp
from jax.experimental import pallas as pl
from jax.experimental.pallas import tpu as pltpu

N = 10000
NP = 10240
E = 320000
D = 128
H = 512
T = 65536

_INTERP = False


def _prep_body(hist_ref, x_ref, dinv_ref, y0_ref):
    deg = 1.0 + hist_ref[:N, 0:1] + hist_ref[:N, 1:2]
    dinv = jax.lax.rsqrt(deg)
    dinv_ref[...] = dinv
    y0_ref[...] = x_ref[...] * dinv


def _prep_call(histT, x):
    return pl.pallas_call(
        _prep_body,
        out_shape=(
            jax.ShapeDtypeStruct((N, 1), jnp.float32),
            jax.ShapeDtypeStruct((N, D), jnp.float32),
        ),
        interpret=_INTERP,
    )(histT, x)


def _layer_body(nch, last, parts_ref, yprev_ref, dinv_ref, w_ref, b_ref,
                w4r_ref, o1_ref, o2_ref):
    bn = yprev_ref.shape[0]
    dinv = dinv_ref[...]
    acc = jnp.zeros((bn, H), jnp.float32)
    for q in range(nch):
        zq = (parts_ref[0, q] + parts_ref[1, q]
              + yprev_ref[:, q * 128:(q + 1) * 128]) * dinv
        acc = acc + jnp.dot(zq, w_ref[q * 128:(q + 1) * 128, :],
                            preferred_element_type=jnp.float32)
    h = jnp.tanh(acc + b_ref[...])
    if last:
        o1_ref[...] = h
        o2_ref[...] = h * w4r_ref[...]
    else:
        o1_ref[...] = h * dinv
        o2_ref[...] = h


def _layer_call(parts, yprev, dinv, w, b2d, w4r, last):
    nch = parts.shape[1]
    fin = nch * 128
    bn = 1000
    grid = (N // bn,)
    body = functools.partial(_layer_body, nch, last)
    return pl.pallas_call(
        body,
        grid=grid,
        in_specs=[
            pl.BlockSpec((2, nch, bn, 128), lambda i: (0, 0, i, 0)),
            pl.BlockSpec((bn, fin), lambda i: (i, 0)),
            pl.BlockSpec((bn, 1), lambda i: (i, 0)),
            pl.BlockSpec((fin, H), lambda i: (0, 0)),
            pl.BlockSpec((1, H), lambda i: (0, 0)),
            pl.BlockSpec((1, H), lambda i: (0, 0)),
        ],
        out_specs=(
            pl.BlockSpec((bn, H), lambda i: (i, 0)),
            pl.BlockSpec((bn, H), lambda i: (i, 0)),
        ),
        out_shape=(
            jax.ShapeDtypeStruct((N, H), jnp.float32),
            jax.ShapeDtypeStruct((N, H), jnp.float32),
        ),
        interpret=_INTERP,
    )(parts, yprev, dinv, w, b2d, w4r)


def _head_body(ga_ref, gb_ref, b4_ref, out_ref):
    prod = ga_ref[...] * gb_ref[...]
    out_ref[...] = jnp.sum(prod, axis=1, keepdims=True) + b4_ref[...]


def _head_call(ga, gb, b4_2d):
    bt = 2048
    return pl.pallas_call(
        _head_body,
        grid=(T // bt,),
        in_specs=[
            pl.BlockSpec((bt, H), lambda i: (i, 0)),
            pl.BlockSpec((bt, H), lambda i: (i, 0)),
            pl.BlockSpec((1, 1), lambda i: (0, 0)),
        ],
        out_specs=pl.BlockSpec((bt, 1), lambda i: (i, 0)),
        out_shape=jax.ShapeDtypeStruct((T, 1), jnp.float32),
        interpret=_INTERP,
    )(ga, gb, b4_2d)


def _tmp_hist(dst):
    indeg = jax.ops.segment_sum(jnp.ones((E,), jnp.float32), dst, num_segments=N)
    histT = jnp.zeros((NP, 2), jnp.float32).at[:N, 0].set(indeg)
    return histT


def _tmp_agg(y, src, dst, nch):
    s = jax.ops.segment_sum(y[src], dst, num_segments=N)
    sq = s.reshape(N, nch, 128).transpose(1, 0, 2)
    parts = jnp.zeros((2, nch, NP, 128), jnp.float32).at[0, :, :N].set(sq)
    return parts


def kernel(x, edge_index, train_edge_id, W1, b1, W2, b2, W3, b3, W4, b4):
    src = edge_index[0]
    dst = edge_index[1]

    histT = _tmp_hist(dst)
    dinv, y0 = _prep_call(histT, x)

    w4r = W4.reshape(1, H)
    b1r = b1.reshape(1, H)
    b2r = b2.reshape(1, H)
    b3r = b3.reshape(1, H)

    parts1 = _tmp_agg(y0, src, dst, 1)
    y1, _ = _layer_call(parts1, y0, dinv, W1, b1r, w4r, last=False)

    parts2 = _tmp_agg(y1, src, dst, 4)
    y2, _ = _layer_call(parts2, y1, dinv, W2, b2r, w4r, last=False)

    parts3 = _tmp_agg(y2, src, dst, 4)
    h3, h3w = _layer_call(parts3, y2, dinv, W3, b3r, w4r, last=True)

    node_a = src[train_edge_id]
    node_b = dst[train_edge_id]
    ga = h3[node_a]
    gb = h3w[node_b]
    return _head_call(ga, gb, b4.reshape(1, 1))


# speedup vs baseline: 6.1384x; 2.4124x over previous
import functools

import jax
import jax.numpy as jnp
from jax.experimental import pallas as pl
from jax.experimental.pallas import tpu as pltpu
from jax.experimental.pallas import tpu_sc as plsc

N = 10000
NP = 10240
E = 320000
D = 128
H = 512
T = 65536

_INTERP = False


def _prep_body(hist_ref, x_ref, dinv_ref, y0_ref):
    deg = 1.0 + hist_ref[0, :N, 0:1] + hist_ref[1, :N, 0:1]
    dinv = jax.lax.rsqrt(deg)
    dinv_ref[...] = dinv
    y0_ref[...] = x_ref[...] * dinv


def _prep_call(histT, x):
    return pl.pallas_call(
        _prep_body,
        out_shape=(
            jax.ShapeDtypeStruct((N, 1), jnp.float32),
            jax.ShapeDtypeStruct((N, D), jnp.float32),
        ),
        interpret=_INTERP,
    )(histT, x)


def _layer_body(nch, last, parts_ref, yprev_ref, dinv_ref, w_ref, b_ref,
                w4r_ref, o1_ref, o2_ref):
    bn = yprev_ref.shape[0]
    dinv = dinv_ref[...]
    acc = jnp.zeros((bn, H), jnp.float32)
    for q in range(nch):
        zq = (parts_ref[0, q] + parts_ref[1, q]
              + yprev_ref[:, q * 128:(q + 1) * 128]) * dinv
        acc = acc + jnp.dot(zq, w_ref[q * 128:(q + 1) * 128, :],
                            preferred_element_type=jnp.float32)
    h = jnp.tanh(acc + b_ref[...])
    if last:
        o1_ref[...] = h
        o2_ref[...] = h * w4r_ref[...]
    else:
        o1_ref[...] = h * dinv
        o2_ref[...] = h


def _layer_call(parts, yprev, dinv, w, b2d, w4r, last):
    nch = parts.shape[1]
    fin = nch * 128
    bn = 1000
    grid = (N // bn,)
    body = functools.partial(_layer_body, nch, last)
    return pl.pallas_call(
        body,
        grid=grid,
        in_specs=[
            pl.BlockSpec((2, nch, bn, 128), lambda i: (0, 0, i, 0)),
            pl.BlockSpec((bn, fin), lambda i: (i, 0)),
            pl.BlockSpec((bn, 1), lambda i: (i, 0)),
            pl.BlockSpec((fin, H), lambda i: (0, 0)),
            pl.BlockSpec((1, H), lambda i: (0, 0)),
            pl.BlockSpec((1, H), lambda i: (0, 0)),
        ],
        out_specs=(
            pl.BlockSpec((bn, H), lambda i: (i, 0)),
            pl.BlockSpec((bn, H), lambda i: (i, 0)),
        ),
        out_shape=(
            jax.ShapeDtypeStruct((N, H), jnp.float32),
            jax.ShapeDtypeStruct((N, H), jnp.float32),
        ),
        interpret=_INTERP,
    )(parts, yprev, dinv, w, b2d, w4r)


def _head_body(ga_ref, gb_ref, b4_ref, out_ref):
    prod = ga_ref[...] * gb_ref[...]
    out_ref[...] = jnp.sum(prod, axis=1, keepdims=True) + b4_ref[...]


def _head_call(ga, gb, b4_2d):
    bt = 2048
    return pl.pallas_call(
        _head_body,
        grid=(T // bt,),
        in_specs=[
            pl.BlockSpec((bt, H), lambda i: (i, 0)),
            pl.BlockSpec((bt, H), lambda i: (i, 0)),
            pl.BlockSpec((1, 1), lambda i: (0, 0)),
        ],
        out_specs=pl.BlockSpec((bt, 1), lambda i: (i, 0)),
        out_shape=jax.ShapeDtypeStruct((T, 1), jnp.float32),
        interpret=_INTERP,
    )(ga, gb, b4_2d)


_SC_MESH = plsc.VectorSubcoreMesh(core_axis_name="c", subcore_axis_name="s")
EB = 80
EW = E // 32
NROW = NP // 16


def _deg_call(dst):
    RW = 128
    zeros = jnp.zeros((NROW, RW), jnp.float32)
    ones = jnp.ones((EB, RW), jnp.float32)

    @functools.partial(
        pl.kernel,
        out_type=jax.ShapeDtypeStruct((2, NP, RW), jnp.float32),
        mesh=_SC_MESH,
        scratch_types=[
            pltpu.VMEM((1, EB), jnp.int32),
            pltpu.VMEM((EB, RW), jnp.float32),
            pltpu.VMEM_SHARED((NP, RW), jnp.float32),
        ],
    )
    def k(dst_hbm, zeros_hbm, ones_hbm, out_hbm, dst_v, ones_v, shared):
        c = jax.lax.axis_index("c")
        s = jax.lax.axis_index("s")
        pltpu.sync_copy(zeros_hbm, shared.at[pl.ds(s * NROW, NROW)])
        pltpu.sync_copy(ones_hbm, ones_v)
        plsc.subcore_barrier()
        base = (c * 16 + s) * EW

        @pl.loop(0, EW // EB)
        def _(i):
            pltpu.sync_copy(dst_hbm.at[pl.ds(base + i * EB, EB)], dst_v.at[0])
            pltpu.sync_copy(ones_v, shared.at[dst_v.at[0]], add=True)

        plsc.subcore_barrier()
        pltpu.sync_copy(shared.at[pl.ds(s * NROW, NROW)],
                        out_hbm.at[c, pl.ds(s * NROW, NROW)])

    return k(dst, zeros, ones)


def _agg_call(table, src, dst, nch):
    zeros = jnp.zeros((NROW, 128), jnp.float32)
    half = E // 2

    @functools.partial(
        pl.kernel,
        out_type=jax.ShapeDtypeStruct((2, nch, NP, 128), jnp.float32),
        mesh=_SC_MESH,
        scratch_types=[
            pltpu.VMEM((EB,), jnp.int32),
            pltpu.VMEM((EB,), jnp.int32),
            pltpu.VMEM((1, EB), jnp.int32),
            pltpu.VMEM((EB, 128), jnp.float32),
            pltpu.VMEM_SHARED((NP, 128), jnp.float32),
        ],
    )
    def k(table_hbm, src_hbm, dst_hbm, zeros_hbm, out_hbm,
          src_v, idx_v, dst_v, rows_v, shared):
        c = jax.lax.axis_index("c")
        s = jax.lax.axis_index("s")
        base = c * half + s * (half // 16)

        for q in range(nch):
            pltpu.sync_copy(zeros_hbm, shared.at[pl.ds(s * NROW, NROW)])
            plsc.subcore_barrier()

            @pl.loop(0, half // 16 // EB)
            def _(i):
                eb = base + i * EB
                pltpu.sync_copy(src_hbm.at[pl.ds(eb, EB)], src_v)
                pltpu.sync_copy(dst_hbm.at[pl.ds(eb, EB)], dst_v.at[0])
                for t in range(EB // 16):
                    sl = pl.ds(t * 16, 16)
                    idx_v[sl] = src_v[sl] * nch + q
                pltpu.sync_copy(table_hbm.at[idx_v], rows_v)
                pltpu.sync_copy(rows_v, shared.at[dst_v.at[0]], add=True)

            plsc.subcore_barrier()
            pltpu.sync_copy(shared.at[pl.ds(s * NROW, NROW)],
                            out_hbm.at[c, q, pl.ds(s * NROW, NROW)])
            plsc.subcore_barrier()

    return k(table, src, dst, zeros)


def _tmp_agg(y, src, dst, nch):
    s = jax.ops.segment_sum(y[src], dst, num_segments=N)
    sq = s.reshape(N, nch, 128).transpose(1, 0, 2)
    parts = jnp.zeros((2, nch, NP, 128), jnp.float32).at[0, :, :N].set(sq)
    return parts


def kernel(x, edge_index, train_edge_id, W1, b1, W2, b2, W3, b3, W4, b4):
    src = edge_index[0]
    dst = edge_index[1]

    hist = _deg_call(dst)
    dinv, y0 = _prep_call(hist, x)

    w4r = W4.reshape(1, H)
    b1r = b1.reshape(1, H)
    b2r = b2.reshape(1, H)
    b3r = b3.reshape(1, H)

    parts1 = _agg_call(y0, src, dst, 1)
    y1, _ = _layer_call(parts1, y0, dinv, W1, b1r, w4r, last=False)

    parts2 = _agg_call(y1.reshape(N * 4, 128), src, dst, 4)
    y2, _ = _layer_call(parts2, y1, dinv, W2, b2r, w4r, last=False)

    parts3 = _agg_call(y2.reshape(N * 4, 128), src, dst, 4)
    h3, h3w = _layer_call(parts3, y2, dinv, W3, b3r, w4r, last=True)

    node_a = src[train_edge_id]
    node_b = dst[train_edge_id]
    ga = h3[node_a]
    gb = h3w[node_b]
    return _head_call(ga, gb, b4.reshape(1, 1))
